# baseline (speedup 1.0000x reference)
import functools

import jax
import jax.numpy as jnp
from jax import lax
from jax.experimental import pallas as pl
from jax.experimental.pallas import tpu as pltpu
from jax.experimental.pallas import tpu_sc as plsc

LANES = 16
NC = 2
NS = 16
CHUNK = 128


def _sc_edge_weights(edge_index, n_nodes):
    E = edge_index.shape[1]
    NCH = E // CHUNK
    B1 = NCH // NS
    R1 = NCH - B1 * NS
    P1M = B1 * CHUNK // LANES
    PX = CHUNK // LANES
    B2 = NCH // (NC * NS)
    C0LEN = B2 * CHUNK
    P2M = C0LEN // LANES
    EMAX = (B1 + 1) * CHUNK
    NPAD = -(-n_nodes // (NS * LANES)) * (NS * LANES)
    NV = NPAD // LANES
    STRIPE = NPAD // NS
    SV = STRIPE // LANES

    mesh = plsc.VectorSubcoreMesh(core_axis_name="c", subcore_axis_name="s")

    @functools.partial(
        pl.kernel,
        out_type=(
            jax.ShapeDtypeStruct((NC * NS, NPAD), jnp.float32),
            jax.ShapeDtypeStruct((NPAD,), jnp.float32),
        ),
        mesh=mesh,
        scratch_types=[
            pltpu.VMEM((2, EMAX), jnp.int32),
            pltpu.VMEM((NPAD,), jnp.float32),
            pltpu.VMEM((NPAD,), jnp.float32),
            pltpu.VMEM((NPAD,), jnp.float32),
            pltpu.VMEM((NS, 2, STRIPE), jnp.float32),
            pltpu.VMEM((2, STRIPE), jnp.float32),
            pltpu.VMEM_SHARED((NS, 2, NPAD), jnp.float32),
            pltpu.VMEM_SHARED((NPAD,), jnp.float32),
            pltpu.SemaphoreType.DMA,
        ],
        compiler_params=pltpu.CompilerParams(needs_layout_passes=False),
    )
    def ew_kernel(edge_hbm, wp_hbm, degsrc_hbm, ed_v, degs_v, degd_v,
                  w_v, peer_v, sred_v, stage_sh, rdst_sh, sem):
        c = lax.axis_index("c")
        t = lax.axis_index("s")
        zeros = jnp.zeros((LANES,), jnp.float32)
        ones = jnp.full((LANES,), 1.0, jnp.float32)

        off1 = (B1 * t + jnp.minimum(t, R1)) * CHUNK
        ld = pltpu.async_copy(edge_hbm.at[:, pl.ds(off1, B1 * CHUNK)],
                              ed_v.at[:, pl.ds(0, B1 * CHUNK)], sem)
        extra = t < R1

        @pl.when(extra)
        def _():
            ldx = pltpu.async_copy(
                edge_hbm.at[:, pl.ds(off1 + B1 * CHUNK, CHUNK)],
                ed_v.at[:, pl.ds(B1 * CHUNK, CHUNK)], sem)
            ldx.wait()

        @plsc.parallel_loop(0, NV, unroll=8)
        def zero_body(i):
            sl = pl.ds(i * LANES, LANES)
            degs_v[sl] = zeros
            degd_v[sl] = zeros
            w_v[sl] = zeros

        ld.wait()

        @plsc.parallel_loop(0, P1M, unroll=8)
        def p1_body(i):
            sl = pl.ds(i * LANES, LANES)
            plsc.addupdate_scatter(degs_v, [ed_v[0, sl]], ones)
            plsc.addupdate_scatter(degd_v, [ed_v[1, sl]], ones)

        @pl.when(extra)
        def _():
            @plsc.parallel_loop(0, PX, unroll=8)
            def p1x_body(i):
                sl = pl.ds(P1M * LANES + i * LANES, LANES)
                plsc.addupdate_scatter(degs_v, [ed_v[0, sl]], ones)
                plsc.addupdate_scatter(degd_v, [ed_v[1, sl]], ones)

        st_s = pltpu.async_copy(degs_v, stage_sh.at[t, 0], sem)
        st_d = pltpu.async_copy(degd_v, stage_sh.at[t, 1], sem)
        st_s.wait()
        st_d.wait()
        plsc.subcore_barrier()

        fetches = []
        for tt in range(NS):
            for a in range(2):
                fetches.append(pltpu.async_copy(
                    stage_sh.at[tt, a, pl.ds(t * STRIPE, STRIPE)],
                    peer_v.at[tt, a], sem))
        for f in fetches:
            f.wait()

        @plsc.parallel_loop(0, SV, unroll=4)
        def acc_body(j):
            for a in range(2):
                sl = pl.ds(j * LANES, LANES)
                v = peer_v[0, a, sl]
                for tt in range(1, NS):
                    v = v + peer_v[tt, a, sl]
                sred_v[a, sl] = v

        @pl.when(c == 0)
        def _():
            wb = pltpu.async_copy(
                sred_v.at[0], degsrc_hbm.at[pl.ds(t * STRIPE, STRIPE)], sem)
            wb.wait()

        magic = jnp.full((LANES,), 0x5F3759DF, jnp.int32)
        half = jnp.full((LANES,), 0.5, jnp.float32)
        th = jnp.full((LANES,), 1.5, jnp.float32)

        @plsc.parallel_loop(0, SV, unroll=4)
        def rs_body(j):
            sl = pl.ds(j * LANES, LANES)
            v = jnp.maximum(sred_v[1, sl], ones)
            y = plsc.bitcast(magic - (plsc.bitcast(v, jnp.int32) >> 1),
                             jnp.float32)
            y = y * (th - half * v * y * y)
            y = y * (th - half * v * y * y)
            y = y * (th - half * v * y * y)
            sred_v[1, sl] = y

        wb1 = pltpu.async_copy(sred_v.at[1],
                               rdst_sh.at[pl.ds(t * STRIPE, STRIPE)], sem)
        wb1.wait()
        plsc.subcore_barrier()

        rb = pltpu.async_copy(rdst_sh, degd_v, sem)
        rb.wait()

        e1 = jnp.where(extra, CHUNK, 0)
        base2 = c * (C0LEN + e1)

        @plsc.parallel_loop(0, P2M, unroll=8)
        def p2_body(i):
            sl = pl.ds(base2 + i * LANES, LANES)
            rd = plsc.load_gather(degd_v, [ed_v[1, sl]])
            plsc.addupdate_scatter(w_v, [ed_v[0, sl]], rd)

        @pl.when(extra & (c == 0))
        def _():
            @plsc.parallel_loop(0, PX, unroll=8)
            def p2x_body(i):
                sl = pl.ds(C0LEN + i * LANES, LANES)
                rd = plsc.load_gather(degd_v, [ed_v[1, sl]])
                plsc.addupdate_scatter(w_v, [ed_v[0, sl]], rd)

        pltpu.sync_copy(w_v, wp_hbm.at[c * NS + t])

    return ew_kernel(edge_index)


def _tc_head(wp, degsrc, x, W, b, Wd, bd):
    N, D = x.shape
    L = Wd.shape[1]

    def body(wp_ref, ds_ref, x_ref, W_ref, b_ref, Wd_ref, bd_ref, o_ref):
        acc = jnp.sum(wp_ref[...], axis=0, keepdims=True)
        r_src = lax.rsqrt(jnp.maximum(ds_ref[...], 1.0))
        w = (acc * r_src)[:, :N]
        t = jnp.sum(w)
        dn = (((1,), (0,)), ((), ()))
        hp = lax.Precision.HIGHEST
        s = lax.dot_general(w, x_ref[...], dn, precision=hp,
                            preferred_element_type=jnp.float32)
        pooled = lax.dot_general(s, W_ref[...], dn, precision=hp,
                                 preferred_element_type=jnp.float32)
        pooled = pooled + t * b_ref[...]
        logits = lax.dot_general(pooled, Wd_ref[...], dn, precision=hp,
                                 preferred_element_type=jnp.float32)
        logits = logits + bd_ref[...]
        e = jnp.exp(logits - jnp.max(logits))
        o_ref[...] = e / jnp.sum(e)

    return pl.pallas_call(
        body,
        out_shape=jax.ShapeDtypeStruct((1, L), jnp.float32),
    )(wp, degsrc.reshape(1, -1), x, W, b.reshape(1, D), Wd, bd.reshape(1, L))


def kernel(x, edge_index, W, b, Wd, bd):
    wp, degsrc = _sc_edge_weights(edge_index, x.shape[0])
    out = _tc_head(wp, degsrc, x, W, b, Wd, bd)
    return out.reshape(-1)


# speedup vs baseline: 1.0463x; 1.0463x over previous
import functools

import jax
import jax.numpy as jnp
from jax import lax
from jax.experimental import pallas as pl
from jax.experimental.pallas import tpu as pltpu
from jax.experimental.pallas import tpu_sc as plsc

LANES = 16
NC = 2
NS = 16
CHUNK = 128


def _sc_edge_weights(edge_index, n_nodes):
    E = edge_index.shape[1]
    NCH = E // CHUNK
    B1 = NCH // NS
    R1 = NCH - B1 * NS
    P1M = B1 * CHUNK // LANES
    PX = CHUNK // LANES
    B2 = NCH // (NC * NS)
    C0LEN = B2 * CHUNK
    P2M = C0LEN // LANES
    EMAX = (B1 + 1) * CHUNK
    NPAD = -(-n_nodes // (NS * LANES)) * (NS * LANES)
    NV = NPAD // LANES
    STRIPE = NPAD // NS
    SV = STRIPE // LANES

    mesh = plsc.VectorSubcoreMesh(core_axis_name="c", subcore_axis_name="s")

    @functools.partial(
        pl.kernel,
        out_type=(
            jax.ShapeDtypeStruct((NC * NS, NPAD), jnp.float32),
            jax.ShapeDtypeStruct((NPAD,), jnp.float32),
        ),
        mesh=mesh,
        scratch_types=[
            pltpu.VMEM((2, EMAX), jnp.int32),
            pltpu.VMEM((NPAD,), jnp.float32),
            pltpu.VMEM((NPAD,), jnp.float32),
            pltpu.VMEM((NPAD,), jnp.float32),
            pltpu.VMEM((NS, 2, STRIPE), jnp.float32),
            pltpu.VMEM((2, STRIPE), jnp.float32),
            pltpu.VMEM_SHARED((NS, 2, NPAD), jnp.float32),
            pltpu.VMEM_SHARED((NPAD,), jnp.float32),
            pltpu.SemaphoreType.DMA,
            pltpu.SemaphoreType.DMA,
        ],
        compiler_params=pltpu.CompilerParams(needs_layout_passes=False),
    )
    def ew_kernel(edge_hbm, wp_hbm, degsrc_hbm, ed_v, degs_v, degd_v,
                  w_v, peer_v, sred_v, stage_sh, rdst_sh, sem, sem2):
        c = lax.axis_index("c")
        t = lax.axis_index("s")
        zeros = jnp.zeros((LANES,), jnp.float32)
        ones = jnp.full((LANES,), 1.0, jnp.float32)

        off1 = (B1 * t + jnp.minimum(t, R1)) * CHUNK
        HALF = (B1 // 2) * CHUNK
        ld_a = pltpu.async_copy(edge_hbm.at[:, pl.ds(off1, HALF)],
                                ed_v.at[:, pl.ds(0, HALF)], sem)
        ld_b = pltpu.async_copy(
            edge_hbm.at[:, pl.ds(off1 + HALF, B1 * CHUNK - HALF)],
            ed_v.at[:, pl.ds(HALF, B1 * CHUNK - HALF)], sem2)
        extra = t < R1

        @pl.when(extra)
        def _():
            ldx = pltpu.async_copy(
                edge_hbm.at[:, pl.ds(off1 + B1 * CHUNK, CHUNK)],
                ed_v.at[:, pl.ds(B1 * CHUNK, CHUNK)], sem)
            ldx.wait()

        @plsc.parallel_loop(0, NV, unroll=8)
        def zero_body(i):
            sl = pl.ds(i * LANES, LANES)
            degs_v[sl] = zeros
            degd_v[sl] = zeros
            w_v[sl] = zeros

        ld_a.wait()

        P1H = HALF // LANES

        @plsc.parallel_loop(0, P1H, unroll=8)
        def p1a_body(i):
            sl = pl.ds(i * LANES, LANES)
            plsc.addupdate_scatter(degs_v, [ed_v[0, sl]], ones)
            plsc.addupdate_scatter(degd_v, [ed_v[1, sl]], ones)

        ld_b.wait()

        @plsc.parallel_loop(0, P1M - P1H, unroll=8)
        def p1b_body(i):
            sl = pl.ds(HALF + i * LANES, LANES)
            plsc.addupdate_scatter(degs_v, [ed_v[0, sl]], ones)
            plsc.addupdate_scatter(degd_v, [ed_v[1, sl]], ones)

        @pl.when(extra)
        def _():
            @plsc.parallel_loop(0, PX, unroll=8)
            def p1x_body(i):
                sl = pl.ds(P1M * LANES + i * LANES, LANES)
                plsc.addupdate_scatter(degs_v, [ed_v[0, sl]], ones)
                plsc.addupdate_scatter(degd_v, [ed_v[1, sl]], ones)

        st_s = pltpu.async_copy(degs_v, stage_sh.at[t, 0], sem)
        st_d = pltpu.async_copy(degd_v, stage_sh.at[t, 1], sem)
        st_s.wait()
        st_d.wait()
        plsc.subcore_barrier()

        fetches = []
        for tt in range(NS):
            for a in range(2):
                fetches.append(pltpu.async_copy(
                    stage_sh.at[tt, a, pl.ds(t * STRIPE, STRIPE)],
                    peer_v.at[tt, a], sem))
        for f in fetches:
            f.wait()

        @plsc.parallel_loop(0, SV, unroll=4)
        def acc_body(j):
            for a in range(2):
                sl = pl.ds(j * LANES, LANES)
                v = peer_v[0, a, sl]
                for tt in range(1, NS):
                    v = v + peer_v[tt, a, sl]
                sred_v[a, sl] = v

        @pl.when(c == 0)
        def _():
            wb = pltpu.async_copy(
                sred_v.at[0], degsrc_hbm.at[pl.ds(t * STRIPE, STRIPE)], sem)
            wb.wait()

        magic = jnp.full((LANES,), 0x5F3759DF, jnp.int32)
        half = jnp.full((LANES,), 0.5, jnp.float32)
        th = jnp.full((LANES,), 1.5, jnp.float32)

        @plsc.parallel_loop(0, SV, unroll=4)
        def rs_body(j):
            sl = pl.ds(j * LANES, LANES)
            v = jnp.maximum(sred_v[1, sl], ones)
            y = plsc.bitcast(magic - (plsc.bitcast(v, jnp.int32) >> 1),
                             jnp.float32)
            y = y * (th - half * v * y * y)
            y = y * (th - half * v * y * y)
            y = y * (th - half * v * y * y)
            sred_v[1, sl] = y

        wb1 = pltpu.async_copy(sred_v.at[1],
                               rdst_sh.at[pl.ds(t * STRIPE, STRIPE)], sem)
        wb1.wait()
        plsc.subcore_barrier()

        rb = pltpu.async_copy(rdst_sh, degd_v, sem)
        rb.wait()

        e1 = jnp.where(extra, CHUNK, 0)
        base2 = c * (C0LEN + e1)

        @plsc.parallel_loop(0, P2M, unroll=8)
        def p2_body(i):
            sl = pl.ds(base2 + i * LANES, LANES)
            rd = plsc.load_gather(degd_v, [ed_v[1, sl]])
            plsc.addupdate_scatter(w_v, [ed_v[0, sl]], rd)

        @pl.when(extra & (c == 0))
        def _():
            @plsc.parallel_loop(0, PX, unroll=8)
            def p2x_body(i):
                sl = pl.ds(C0LEN + i * LANES, LANES)
                rd = plsc.load_gather(degd_v, [ed_v[1, sl]])
                plsc.addupdate_scatter(w_v, [ed_v[0, sl]], rd)

        pltpu.sync_copy(w_v, wp_hbm.at[c * NS + t])

    return ew_kernel(edge_index)


def _tc_head(wp, degsrc, x, W, b, Wd, bd):
    N, D = x.shape
    L = Wd.shape[1]

    def body(wp_ref, ds_ref, x_ref, W_ref, b_ref, Wd_ref, bd_ref, o_ref):
        acc = jnp.sum(wp_ref[...], axis=0, keepdims=True)
        r_src = lax.rsqrt(jnp.maximum(ds_ref[...], 1.0))
        w = (acc * r_src)[:, :N]
        t = jnp.sum(w)
        dn = (((1,), (0,)), ((), ()))
        s = lax.dot_general(w, x_ref[...], dn,
                            preferred_element_type=jnp.float32)
        pooled = lax.dot_general(s, W_ref[...], dn,
                                 preferred_element_type=jnp.float32)
        pooled = pooled + t * b_ref[...]
        logits = lax.dot_general(pooled, Wd_ref[...], dn,
                                 preferred_element_type=jnp.float32)
        logits = logits + bd_ref[...]
        e = jnp.exp(logits - jnp.max(logits))
        o_ref[...] = e / jnp.sum(e)

    return pl.pallas_call(
        body,
        out_shape=jax.ShapeDtypeStruct((1, L), jnp.float32),
    )(wp, degsrc.reshape(1, -1), x, W, b.reshape(1, D), Wd, bd.reshape(1, L))


def kernel(x, edge_index, W, b, Wd, bd):
    wp, degsrc = _sc_edge_weights(edge_index, x.shape[0])
    out = _tc_head(wp, degsrc, x, W, b, Wd, bd)
    return out.reshape(-1)


# speedup vs baseline: 1.0589x; 1.0121x over previous
import functools

import jax
import jax.numpy as jnp
from jax import lax
from jax.experimental import pallas as pl
from jax.experimental.pallas import tpu as pltpu
from jax.experimental.pallas import tpu_sc as plsc

LANES = 16
NC = 2
NS = 16
CHUNK = 128


def _sc_edge_weights(edge_index, n_nodes):
    E = edge_index.shape[1]
    NCH = E // CHUNK
    B1 = NCH // NS
    R1 = NCH - B1 * NS
    P1M = B1 * CHUNK // LANES
    PX = CHUNK // LANES
    K0 = B1 // 4
    P2A = K0 * CHUNK // LANES
    P2B = (B1 - K0) * CHUNK // LANES
    EMAX = (B1 + 1) * CHUNK
    NPAD = -(-n_nodes // (NS * LANES)) * (NS * LANES)
    NV = NPAD // LANES
    STRIPE = NPAD // NS
    SV = STRIPE // LANES

    mesh = plsc.VectorSubcoreMesh(core_axis_name="c", subcore_axis_name="s")

    @functools.partial(
        pl.kernel,
        out_type=(
            jax.ShapeDtypeStruct((NC * NS, NPAD), jnp.float32),
            jax.ShapeDtypeStruct((NPAD,), jnp.float32),
        ),
        mesh=mesh,
        scratch_types=[
            pltpu.VMEM((2, EMAX), jnp.int32),
            pltpu.VMEM((NPAD,), jnp.float32),
            pltpu.VMEM((NPAD,), jnp.float32),
            pltpu.VMEM((NPAD,), jnp.float32),
            pltpu.VMEM((NS, 2, STRIPE), jnp.float32),
            pltpu.VMEM((2, STRIPE), jnp.float32),
            pltpu.VMEM_SHARED((NS, 2, NPAD), jnp.float32),
            pltpu.VMEM_SHARED((NPAD,), jnp.float32),
            pltpu.SemaphoreType.DMA,
            pltpu.SemaphoreType.DMA,
        ],
        compiler_params=pltpu.CompilerParams(needs_layout_passes=False),
    )
    def ew_kernel(edge_hbm, wp_hbm, degsrc_hbm, ed_v, degs_v, degd_v,
                  w_v, peer_v, sred_v, stage_sh, rdst_sh, sem, sem2):
        c = lax.axis_index("c")
        t = lax.axis_index("s")
        zeros = jnp.zeros((LANES,), jnp.float32)
        ones = jnp.full((LANES,), 1.0, jnp.float32)

        off1 = (B1 * t + jnp.minimum(t, R1)) * CHUNK
        HALF = (B1 // 2) * CHUNK
        ld_a = pltpu.async_copy(edge_hbm.at[:, pl.ds(off1, HALF)],
                                ed_v.at[:, pl.ds(0, HALF)], sem)
        ld_b = pltpu.async_copy(
            edge_hbm.at[:, pl.ds(off1 + HALF, B1 * CHUNK - HALF)],
            ed_v.at[:, pl.ds(HALF, B1 * CHUNK - HALF)], sem2)
        extra = t < R1

        @pl.when(extra)
        def _():
            ldx = pltpu.async_copy(
                edge_hbm.at[:, pl.ds(off1 + B1 * CHUNK, CHUNK)],
                ed_v.at[:, pl.ds(B1 * CHUNK, CHUNK)], sem)
            ldx.wait()

        @plsc.parallel_loop(0, NV, unroll=8)
        def zero_body(i):
            sl = pl.ds(i * LANES, LANES)
            degd_v[sl] = zeros
            w_v[sl] = zeros

        @pl.when(c == 0)
        def _():
            @plsc.parallel_loop(0, NV, unroll=8)
            def zs_body(i):
                degs_v[pl.ds(i * LANES, LANES)] = zeros

        ld_a.wait()

        P1H = HALF // LANES

        @pl.when(c == 0)
        def _():
            @plsc.parallel_loop(0, P1H, unroll=8)
            def p1a_body(i):
                sl = pl.ds(i * LANES, LANES)
                plsc.addupdate_scatter(degs_v, [ed_v[0, sl]], ones)
                plsc.addupdate_scatter(degd_v, [ed_v[1, sl]], ones)

        @pl.when(c != 0)
        def _():
            @plsc.parallel_loop(0, P1H, unroll=8)
            def p1a1_body(i):
                sl = pl.ds(i * LANES, LANES)
                plsc.addupdate_scatter(degd_v, [ed_v[1, sl]], ones)

        ld_b.wait()

        @pl.when(c == 0)
        def _():
            @plsc.parallel_loop(0, P1M - P1H, unroll=8)
            def p1b_body(i):
                sl = pl.ds(HALF + i * LANES, LANES)
                plsc.addupdate_scatter(degs_v, [ed_v[0, sl]], ones)
                plsc.addupdate_scatter(degd_v, [ed_v[1, sl]], ones)

            @pl.when(extra)
            def _():
                @plsc.parallel_loop(0, PX, unroll=8)
                def p1x_body(i):
                    sl = pl.ds(P1M * LANES + i * LANES, LANES)
                    plsc.addupdate_scatter(degs_v, [ed_v[0, sl]], ones)
                    plsc.addupdate_scatter(degd_v, [ed_v[1, sl]], ones)

        @pl.when(c != 0)
        def _():
            @plsc.parallel_loop(0, P1M - P1H, unroll=8)
            def p1b1_body(i):
                sl = pl.ds(HALF + i * LANES, LANES)
                plsc.addupdate_scatter(degd_v, [ed_v[1, sl]], ones)

            @pl.when(extra)
            def _():
                @plsc.parallel_loop(0, PX, unroll=8)
                def p1x1_body(i):
                    sl = pl.ds(P1M * LANES + i * LANES, LANES)
                    plsc.addupdate_scatter(degd_v, [ed_v[1, sl]], ones)

        st_d = pltpu.async_copy(degd_v, stage_sh.at[t, 1], sem)
        st_d.wait()

        @pl.when(c == 0)
        def _():
            st_s = pltpu.async_copy(degs_v, stage_sh.at[t, 0], sem)
            st_s.wait()

        plsc.subcore_barrier()

        fetches = []
        for tt in range(NS):
            fetches.append(pltpu.async_copy(
                stage_sh.at[tt, 1, pl.ds(t * STRIPE, STRIPE)],
                peer_v.at[tt, 1], sem))
        for f in fetches:
            f.wait()

        @plsc.parallel_loop(0, SV, unroll=4)
        def acc_body(j):
            sl = pl.ds(j * LANES, LANES)
            v = peer_v[0, 1, sl]
            for tt in range(1, NS):
                v = v + peer_v[tt, 1, sl]
            sred_v[1, sl] = v

        @pl.when(c == 0)
        def _():
            sfetch = []
            for tt in range(NS):
                sfetch.append(pltpu.async_copy(
                    stage_sh.at[tt, 0, pl.ds(t * STRIPE, STRIPE)],
                    peer_v.at[tt, 0], sem))
            for f in sfetch:
                f.wait()

            @plsc.parallel_loop(0, SV, unroll=4)
            def sacc_body(j):
                sl = pl.ds(j * LANES, LANES)
                v = peer_v[0, 0, sl]
                for tt in range(1, NS):
                    v = v + peer_v[tt, 0, sl]
                sred_v[0, sl] = v

            wb = pltpu.async_copy(
                sred_v.at[0], degsrc_hbm.at[pl.ds(t * STRIPE, STRIPE)], sem)
            wb.wait()

        magic = jnp.full((LANES,), 0x5F3759DF, jnp.int32)
        half = jnp.full((LANES,), 0.5, jnp.float32)
        th = jnp.full((LANES,), 1.5, jnp.float32)

        @plsc.parallel_loop(0, SV, unroll=4)
        def rs_body(j):
            sl = pl.ds(j * LANES, LANES)
            v = jnp.maximum(sred_v[1, sl], ones)
            y = plsc.bitcast(magic - (plsc.bitcast(v, jnp.int32) >> 1),
                             jnp.float32)
            y = y * (th - half * v * y * y)
            y = y * (th - half * v * y * y)
            y = y * (th - half * v * y * y)
            sred_v[1, sl] = y

        wb1 = pltpu.async_copy(sred_v.at[1],
                               rdst_sh.at[pl.ds(t * STRIPE, STRIPE)], sem)
        wb1.wait()
        plsc.subcore_barrier()

        rb = pltpu.async_copy(rdst_sh, degd_v, sem)
        rb.wait()

        @pl.when(c == 0)
        def _():
            @plsc.parallel_loop(0, P2A, unroll=8)
            def p2a_body(i):
                sl = pl.ds(i * LANES, LANES)
                rd = plsc.load_gather(degd_v, [ed_v[1, sl]])
                plsc.addupdate_scatter(w_v, [ed_v[0, sl]], rd)

        @pl.when(c != 0)
        def _():
            @plsc.parallel_loop(0, P2B, unroll=8)
            def p2b_body(i):
                sl = pl.ds(K0 * CHUNK + i * LANES, LANES)
                rd = plsc.load_gather(degd_v, [ed_v[1, sl]])
                plsc.addupdate_scatter(w_v, [ed_v[0, sl]], rd)

            @pl.when(extra)
            def _():
                @plsc.parallel_loop(0, PX, unroll=8)
                def p2x_body(i):
                    sl = pl.ds(B1 * CHUNK + i * LANES, LANES)
                    rd = plsc.load_gather(degd_v, [ed_v[1, sl]])
                    plsc.addupdate_scatter(w_v, [ed_v[0, sl]], rd)

        pltpu.sync_copy(w_v, wp_hbm.at[c * NS + t])

    return ew_kernel(edge_index)


def _tc_head(wp, degsrc, x, W, b, Wd, bd):
    N, D = x.shape
    L = Wd.shape[1]

    def body(wp_ref, ds_ref, x_ref, W_ref, b_ref, Wd_ref, bd_ref, o_ref):
        acc = jnp.sum(wp_ref[...], axis=0, keepdims=True)
        r_src = lax.rsqrt(jnp.maximum(ds_ref[...], 1.0))
        w = (acc * r_src)[:, :N]
        t = jnp.sum(w)
        dn = (((1,), (0,)), ((), ()))
        s = lax.dot_general(w, x_ref[...], dn,
                            preferred_element_type=jnp.float32)
        pooled = lax.dot_general(s, W_ref[...], dn,
                                 preferred_element_type=jnp.float32)
        pooled = pooled + t * b_ref[...]
        logits = lax.dot_general(pooled, Wd_ref[...], dn,
                                 preferred_element_type=jnp.float32)
        logits = logits + bd_ref[...]
        e = jnp.exp(logits - jnp.max(logits))
        o_ref[...] = e / jnp.sum(e)

    return pl.pallas_call(
        body,
        out_shape=jax.ShapeDtypeStruct((1, L), jnp.float32),
    )(wp, degsrc.reshape(1, -1), x, W, b.reshape(1, D), Wd, bd.reshape(1, L))


def kernel(x, edge_index, W, b, Wd, bd):
    wp, degsrc = _sc_edge_weights(edge_index, x.shape[0])
    out = _tc_head(wp, degsrc, x, W, b, Wd, bd)
    return out.reshape(-1)


# speedup vs baseline: 1.0701x; 1.0105x over previous
import functools

import jax
import jax.numpy as jnp
from jax import lax
from jax.experimental import pallas as pl
from jax.experimental.pallas import tpu as pltpu
from jax.experimental.pallas import tpu_sc as plsc

LANES = 16
NC = 2
NS = 16
CHUNK = 128


def _sc_edge_weights(edge_index, n_nodes):
    E = edge_index.shape[1]
    NCH = E // CHUNK
    B1 = NCH // NS
    R1 = NCH - B1 * NS
    P1M = B1 * CHUNK // LANES
    PX = CHUNK // LANES
    K0 = B1 // 4
    P2A = K0 * CHUNK // LANES
    P2B = (B1 - K0) * CHUNK // LANES
    EMAX = (B1 + 1) * CHUNK
    NPAD = -(-n_nodes // (NS * LANES)) * (NS * LANES)
    NV = NPAD // LANES
    STRIPE = NPAD // NS
    SV = STRIPE // LANES

    mesh = plsc.VectorSubcoreMesh(core_axis_name="c", subcore_axis_name="s")

    @functools.partial(
        pl.kernel,
        out_type=(
            jax.ShapeDtypeStruct((NC * NS, NPAD), jnp.float32),
            jax.ShapeDtypeStruct((NPAD,), jnp.float32),
        ),
        mesh=mesh,
        scratch_types=[
            pltpu.VMEM((2, EMAX), jnp.int32),
            pltpu.VMEM((NPAD,), jnp.float32),
            pltpu.VMEM((NPAD,), jnp.float32),
            pltpu.VMEM((NPAD,), jnp.float32),
            pltpu.VMEM((NS, 2, STRIPE), jnp.float32),
            pltpu.VMEM((2, STRIPE), jnp.float32),
            pltpu.VMEM_SHARED((NS, 2, NPAD), jnp.float32),
            pltpu.VMEM_SHARED((NPAD,), jnp.float32),
            pltpu.SemaphoreType.DMA,
            pltpu.SemaphoreType.DMA,
        ],
        compiler_params=pltpu.CompilerParams(needs_layout_passes=False),
    )
    def ew_kernel(edge_hbm, wp_hbm, degsrc_hbm, ed_v, degs_v, degd_v,
                  w_v, peer_v, sred_v, stage_sh, rdst_sh, sem, sem2):
        c = lax.axis_index("c")
        t = lax.axis_index("s")
        zeros = jnp.zeros((LANES,), jnp.float32)
        ones = jnp.full((LANES,), 1.0, jnp.float32)

        off1 = (B1 * t + jnp.minimum(t, R1)) * CHUNK
        HALF = (B1 // 2) * CHUNK
        ld_a = pltpu.async_copy(edge_hbm.at[:, pl.ds(off1, HALF)],
                                ed_v.at[:, pl.ds(0, HALF)], sem)
        ld_b = pltpu.async_copy(
            edge_hbm.at[:, pl.ds(off1 + HALF, B1 * CHUNK - HALF)],
            ed_v.at[:, pl.ds(HALF, B1 * CHUNK - HALF)], sem2)
        extra = t < R1

        @pl.when(extra)
        def _():
            ldx = pltpu.async_copy(
                edge_hbm.at[:, pl.ds(off1 + B1 * CHUNK, CHUNK)],
                ed_v.at[:, pl.ds(B1 * CHUNK, CHUNK)], sem)
            ldx.wait()

        @plsc.parallel_loop(0, NV, unroll=8)
        def zero_body(i):
            sl = pl.ds(i * LANES, LANES)
            degd_v[sl] = zeros
            w_v[sl] = zeros

        @pl.when(c == 0)
        def _():
            @plsc.parallel_loop(0, NV, unroll=8)
            def zs_body(i):
                degs_v[pl.ds(i * LANES, LANES)] = zeros

        ld_a.wait()

        P1H = HALF // LANES

        @pl.when(c == 0)
        def _():
            @plsc.parallel_loop(0, P1H, unroll=8)
            def p1a_body(i):
                sl = pl.ds(i * LANES, LANES)
                plsc.addupdate_scatter(degs_v, [ed_v[0, sl]], ones)
                plsc.addupdate_scatter(degd_v, [ed_v[1, sl]], ones)

        @pl.when(c != 0)
        def _():
            @plsc.parallel_loop(0, P1H, unroll=8)
            def p1a1_body(i):
                sl = pl.ds(i * LANES, LANES)
                plsc.addupdate_scatter(degd_v, [ed_v[1, sl]], ones)

        ld_b.wait()

        @pl.when(c == 0)
        def _():
            @plsc.parallel_loop(0, P1M - P1H, unroll=8)
            def p1b_body(i):
                sl = pl.ds(HALF + i * LANES, LANES)
                plsc.addupdate_scatter(degs_v, [ed_v[0, sl]], ones)
                plsc.addupdate_scatter(degd_v, [ed_v[1, sl]], ones)

            @pl.when(extra)
            def _():
                @plsc.parallel_loop(0, PX, unroll=8)
                def p1x_body(i):
                    sl = pl.ds(P1M * LANES + i * LANES, LANES)
                    plsc.addupdate_scatter(degs_v, [ed_v[0, sl]], ones)
                    plsc.addupdate_scatter(degd_v, [ed_v[1, sl]], ones)

        @pl.when(c != 0)
        def _():
            @plsc.parallel_loop(0, P1M - P1H, unroll=8)
            def p1b1_body(i):
                sl = pl.ds(HALF + i * LANES, LANES)
                plsc.addupdate_scatter(degd_v, [ed_v[1, sl]], ones)

            @pl.when(extra)
            def _():
                @plsc.parallel_loop(0, PX, unroll=8)
                def p1x1_body(i):
                    sl = pl.ds(P1M * LANES + i * LANES, LANES)
                    plsc.addupdate_scatter(degd_v, [ed_v[1, sl]], ones)

        st_d = pltpu.async_copy(degd_v, stage_sh.at[t, 1], sem)
        st_d.wait()

        @pl.when(c == 0)
        def _():
            st_s = pltpu.async_copy(degs_v, stage_sh.at[t, 0], sem)
            st_s.wait()

        plsc.subcore_barrier()

        fetches = []
        for tt in range(NS):
            fetches.append(pltpu.async_copy(
                stage_sh.at[tt, 1, pl.ds(t * STRIPE, STRIPE)],
                peer_v.at[tt, 1], sem))
        for f in fetches:
            f.wait()

        @plsc.parallel_loop(0, SV, unroll=4)
        def acc_body(j):
            sl = pl.ds(j * LANES, LANES)
            v = peer_v[0, 1, sl]
            for tt in range(1, NS):
                v = v + peer_v[tt, 1, sl]
            sred_v[1, sl] = v

        magic = jnp.full((LANES,), 0x5F3759DF, jnp.int32)
        half = jnp.full((LANES,), 0.5, jnp.float32)
        th = jnp.full((LANES,), 1.5, jnp.float32)

        @plsc.parallel_loop(0, SV, unroll=4)
        def rs_body(j):
            sl = pl.ds(j * LANES, LANES)
            v = jnp.maximum(sred_v[1, sl], ones)
            y = plsc.bitcast(magic - (plsc.bitcast(v, jnp.int32) >> 1),
                             jnp.float32)
            y = y * (th - half * v * y * y)
            y = y * (th - half * v * y * y)
            y = y * (th - half * v * y * y)
            sred_v[1, sl] = y

        wb1 = pltpu.async_copy(sred_v.at[1],
                               rdst_sh.at[pl.ds(t * STRIPE, STRIPE)], sem)
        wb1.wait()
        plsc.subcore_barrier()

        rb = pltpu.async_copy(rdst_sh, degd_v, sem)
        rb.wait()

        @pl.when(c == 0)
        def _():
            sfetch = []
            for tt in range(NS):
                sfetch.append(pltpu.async_copy(
                    stage_sh.at[tt, 0, pl.ds(t * STRIPE, STRIPE)],
                    peer_v.at[tt, 0], sem2))

            @plsc.parallel_loop(0, P2A, unroll=8)
            def p2a_body(i):
                sl = pl.ds(i * LANES, LANES)
                rd = plsc.load_gather(degd_v, [ed_v[1, sl]])
                plsc.addupdate_scatter(w_v, [ed_v[0, sl]], rd)

            for f in sfetch:
                f.wait()

            @plsc.parallel_loop(0, SV, unroll=4)
            def sacc_body(j):
                sl = pl.ds(j * LANES, LANES)
                v = peer_v[0, 0, sl]
                for tt in range(1, NS):
                    v = v + peer_v[tt, 0, sl]
                sred_v[0, sl] = v

            wb = pltpu.async_copy(
                sred_v.at[0], degsrc_hbm.at[pl.ds(t * STRIPE, STRIPE)], sem2)
            wb.wait()

        @pl.when(c != 0)
        def _():
            @plsc.parallel_loop(0, P2B, unroll=8)
            def p2b_body(i):
                sl = pl.ds(K0 * CHUNK + i * LANES, LANES)
                rd = plsc.load_gather(degd_v, [ed_v[1, sl]])
                plsc.addupdate_scatter(w_v, [ed_v[0, sl]], rd)

            @pl.when(extra)
            def _():
                @plsc.parallel_loop(0, PX, unroll=8)
                def p2x_body(i):
                    sl = pl.ds(B1 * CHUNK + i * LANES, LANES)
                    rd = plsc.load_gather(degd_v, [ed_v[1, sl]])
                    plsc.addupdate_scatter(w_v, [ed_v[0, sl]], rd)

        pltpu.sync_copy(w_v, wp_hbm.at[c * NS + t])

    return ew_kernel(edge_index)


def _tc_head(wp, degsrc, x, W, b, Wd, bd):
    N, D = x.shape
    L = Wd.shape[1]

    def body(wp_ref, ds_ref, x_ref, W_ref, b_ref, Wd_ref, bd_ref, o_ref):
        acc = jnp.sum(wp_ref[...], axis=0, keepdims=True)
        r_src = lax.rsqrt(jnp.maximum(ds_ref[...], 1.0))
        w = (acc * r_src)[:, :N]
        t = jnp.sum(w)
        dn = (((1,), (0,)), ((), ()))
        s = lax.dot_general(w, x_ref[...], dn,
                            preferred_element_type=jnp.float32)
        pooled = lax.dot_general(s, W_ref[...], dn,
                                 preferred_element_type=jnp.float32)
        pooled = pooled + t * b_ref[...]
        logits = lax.dot_general(pooled, Wd_ref[...], dn,
                                 preferred_element_type=jnp.float32)
        logits = logits + bd_ref[...]
        e = jnp.exp(logits - jnp.max(logits))
        o_ref[...] = e / jnp.sum(e)

    return pl.pallas_call(
        body,
        out_shape=jax.ShapeDtypeStruct((1, L), jnp.float32),
    )(wp, degsrc.reshape(1, -1), x, W, b.reshape(1, D), Wd, bd.reshape(1, L))


def kernel(x, edge_index, W, b, Wd, bd):
    wp, degsrc = _sc_edge_weights(edge_index, x.shape[0])
    out = _tc_head(wp, degsrc, x, W, b, Wd, bd)
    return out.reshape(-1)
